# baseline (speedup 1.0000x reference)
import functools
import math

import jax
import jax.numpy as jnp
from jax.experimental import pallas as pl
from jax.experimental.pallas import tpu as pltpu

_NBUF = 2


def _copy_panel(adj_ref, bufs_ref, sems_ref, panel, slot, bm):
    hm = bm // 2
    pltpu.make_async_copy(
        adj_ref.at[pl.ds(panel * bm, hm), :],
        bufs_ref.at[slot, pl.ds(0, hm), :],
        sems_ref.at[slot, 0],
    ).start()
    pltpu.make_async_copy(
        adj_ref.at[pl.ds(panel * bm + hm, hm), :],
        bufs_ref.at[slot, pl.ds(hm, hm), :],
        sems_ref.at[slot, 1],
    ).start()


def _fused_body(g1, g2, bf, bm, x_ref, w_ref, adj_ref, out_ref,
                sup_ref, bufs_ref, sems_ref):
    g = pl.program_id(0)

    @pl.when(g == 0)
    def _prime_dma():
        for i in range(min(_NBUF, g2)):
            _copy_panel(adj_ref, bufs_ref, sems_ref, i, i, bm)

    @pl.when(g < g1)
    def _support_phase():
        sup = jnp.dot(x_ref[...], w_ref[...],
                      preferred_element_type=jnp.float32)
        sup_ref[pl.ds(g * bf, bf), :] = sup.astype(jnp.bfloat16)

    @pl.when(g >= g1)
    def _spmm_phase():
        m = g - g1
        slot = jax.lax.rem(m, _NBUF)
        hm = bm // 2
        pltpu.make_async_copy(
            adj_ref.at[pl.ds(m * bm, hm), :],
            bufs_ref.at[slot, pl.ds(0, hm), :],
            sems_ref.at[slot, 0],
        ).wait()
        pltpu.make_async_copy(
            adj_ref.at[pl.ds(m * bm + hm, hm), :],
            bufs_ref.at[slot, pl.ds(hm, hm), :],
            sems_ref.at[slot, 1],
        ).wait()
        acc = jnp.dot(bufs_ref[slot], sup_ref[...],
                      preferred_element_type=jnp.float32)
        out_ref[...] = jnp.maximum(acc, 0.0)

        @pl.when(m + _NBUF < g2)
        def _refill():
            _copy_panel(adj_ref, bufs_ref, sems_ref, m + _NBUF, slot, bm)


def kernel(features, adj, weight):
    n, d_in = features.shape
    d_out = weight.shape[1]

    bf = math.gcd(n, 2000)
    bm = math.gcd(n, 400)
    g1 = n // bf
    g2 = n // bm

    body = functools.partial(_fused_body, g1, g2, bf, bm)
    out = pl.pallas_call(
        body,
        grid=(g1 + g2,),
        in_specs=[
            pl.BlockSpec((bf, d_in),
                         lambda g: (jnp.minimum(g, g1 - 1), 0)),
            pl.BlockSpec((d_in, d_out), lambda g: (0, 0)),
            pl.BlockSpec(memory_space=pltpu.MemorySpace.HBM),
        ],
        out_specs=pl.BlockSpec((bm, d_out),
                               lambda g: (jnp.maximum(g - g1, 0), 0)),
        out_shape=jax.ShapeDtypeStruct((n, d_out), jnp.float32),
        scratch_shapes=[
            pltpu.VMEM((n, d_out), jnp.bfloat16),
            pltpu.VMEM((_NBUF, bm, n), jnp.float32),
            pltpu.SemaphoreType.DMA((_NBUF, 2)),
        ],
        compiler_params=pltpu.CompilerParams(
            dimension_semantics=("arbitrary",),
        ),
    )(features, weight, adj)
    return out


# speedup vs baseline: 1.0261x; 1.0261x over previous
import functools
import math

import jax
import jax.numpy as jnp
from jax.experimental import pallas as pl
from jax.experimental.pallas import tpu as pltpu

_NBUF = 3


def _copy_panel(adj_ref, bufs_ref, sems_ref, panel, slot, bm):
    pltpu.make_async_copy(
        adj_ref.at[pl.ds(panel * bm, bm), :],
        bufs_ref.at[slot],
        sems_ref.at[slot],
    ).start()


def _fused_body(g1, g2, bf, bm, x_ref, w_ref, adj_ref, out_ref,
                sup_ref, bufs_ref, sems_ref):
    g = pl.program_id(0)

    @pl.when(g == 0)
    def _prime_dma():
        for i in range(min(_NBUF, g2)):
            _copy_panel(adj_ref, bufs_ref, sems_ref, i, i, bm)

    @pl.when(g < g1)
    def _support_phase():
        sup = jnp.dot(x_ref[...], w_ref[...],
                      preferred_element_type=jnp.float32)
        sup_ref[pl.ds(g * bf, bf), :] = sup.astype(jnp.bfloat16)

    @pl.when(g >= g1)
    def _spmm_phase():
        m = g - g1
        slot = jax.lax.rem(m, _NBUF)
        pltpu.make_async_copy(
            adj_ref.at[pl.ds(m * bm, bm), :],
            bufs_ref.at[slot],
            sems_ref.at[slot],
        ).wait()
        acc = jnp.dot(bufs_ref[slot], sup_ref[...],
                      preferred_element_type=jnp.float32)
        out_ref[...] = jnp.maximum(acc, 0.0)

        @pl.when(m + _NBUF < g2)
        def _refill():
            _copy_panel(adj_ref, bufs_ref, sems_ref, m + _NBUF, slot, bm)


def kernel(features, adj, weight):
    n, d_in = features.shape
    d_out = weight.shape[1]

    bf = math.gcd(n, 2000)
    bm = math.gcd(n, 200)
    g1 = n // bf
    g2 = n // bm

    body = functools.partial(_fused_body, g1, g2, bf, bm)
    out = pl.pallas_call(
        body,
        grid=(g1 + g2,),
        in_specs=[
            pl.BlockSpec((bf, d_in),
                         lambda g: (jnp.minimum(g, g1 - 1), 0)),
            pl.BlockSpec((d_in, d_out), lambda g: (0, 0)),
            pl.BlockSpec(memory_space=pltpu.MemorySpace.HBM),
        ],
        out_specs=pl.BlockSpec((bm, d_out),
                               lambda g: (jnp.maximum(g - g1, 0), 0)),
        out_shape=jax.ShapeDtypeStruct((n, d_out), jnp.float32),
        scratch_shapes=[
            pltpu.VMEM((n, d_out), jnp.bfloat16),
            pltpu.VMEM((_NBUF, bm, n), jnp.float32),
            pltpu.SemaphoreType.DMA((_NBUF,)),
        ],
        compiler_params=pltpu.CompilerParams(
            dimension_semantics=("arbitrary",),
        ),
    )(features, weight, adj)
    return out
